# baseline (speedup 1.0000x reference)
import jax
import jax.numpy as jnp
from jax import lax
from jax.experimental import pallas as pl
from jax.experimental.pallas import tpu as pltpu

GENOMES = 16
FEATURES = 128
EMBED = 16
BATCH = 16384

BT = 2048
PER_G = BATCH // BT
STEPS = GENOMES * PER_G
NBUF = 8
OB = 3


def _embed_kernel(x_hbm, w_ref, o_hbm, *scratch):
    xbufs = scratch[:NBUF]
    obufs = scratch[NBUF:NBUF + OB]
    in_sems = scratch[NBUF + OB:2 * NBUF + OB]
    out_sems = scratch[2 * NBUF + OB:]

    def in_copy(s):
        g, r = divmod(s, PER_G)
        return pltpu.make_async_copy(
            x_hbm.at[g, pl.ds(r * BT, BT), :], xbufs[s % NBUF],
            in_sems[s % NBUF])

    def out_copy(g):
        return pltpu.make_async_copy(
            obufs[g % OB], o_hbm.at[g], out_sems[g % OB])

    for s in range(NBUF):
        in_copy(s).start()

    for s in range(STEPS):
        g, r = divmod(s, PER_G)
        if r == 0 and g >= OB:
            out_copy(g - OB).wait()
        in_copy(s).wait()
        obufs[g % OB][:, r * BT:(r + 1) * BT] = lax.dot_general(
            w_ref[g], xbufs[s % NBUF][...],
            dimension_numbers=(((0,), (1,)), ((), ())),
            preferred_element_type=jnp.float32)
        if r == PER_G - 1:
            out_copy(g).start()
        if s + NBUF < STEPS:
            in_copy(s + NBUF).start()

    for g in range(GENOMES - OB, GENOMES):
        out_copy(g).wait()


def _impl(tensor, W):
    scratch = (
        [pltpu.VMEM((BT, FEATURES), jnp.float32)] * NBUF
        + [pltpu.VMEM((EMBED, BATCH), jnp.float32)] * OB
        + [pltpu.SemaphoreType.DMA] * (NBUF + OB)
    )
    out_t = pl.pallas_call(
        _embed_kernel,
        in_specs=[
            pl.BlockSpec(memory_space=pl.ANY),
            pl.BlockSpec(memory_space=pltpu.VMEM),
        ],
        out_specs=pl.BlockSpec(memory_space=pl.ANY),
        out_shape=jax.ShapeDtypeStruct((GENOMES, EMBED, BATCH), jnp.float32),
        scratch_shapes=scratch,
    )(tensor, W)
    return out_t.transpose(0, 2, 1)


kernel = jax.jit(_impl)


# speedup vs baseline: 1.0044x; 1.0044x over previous
import jax
import jax.numpy as jnp
from jax import lax
from jax.experimental import pallas as pl
from jax.experimental.pallas import tpu as pltpu

GENOMES = 16
FEATURES = 128
EMBED = 16
BATCH = 16384

BT = 2048
PER_G = BATCH // BT
STEPS = GENOMES * PER_G
NBUF = 8
OB = 4


def _embed_kernel(x_hbm, w_ref, o_hbm, *scratch):
    xbufs = scratch[:NBUF]
    obufs = scratch[NBUF:NBUF + OB]
    in_sems = scratch[NBUF + OB:2 * NBUF + OB]
    out_sems = scratch[2 * NBUF + OB:]

    def in_copy(s):
        g, r = divmod(s, PER_G)
        return pltpu.make_async_copy(
            x_hbm.at[g, pl.ds(r * BT, BT), :], xbufs[s % NBUF],
            in_sems[s % NBUF])

    def out_copy(s):
        g, r = divmod(s, PER_G)
        return pltpu.make_async_copy(
            obufs[s % OB], o_hbm.at[g, :, pl.ds(r * BT, BT)],
            out_sems[s % OB])

    for s in range(NBUF):
        in_copy(s).start()

    for s in range(STEPS):
        g = s // PER_G
        if s >= OB:
            out_copy(s - OB).wait()
        in_copy(s).wait()
        obufs[s % OB][...] = lax.dot_general(
            w_ref[g], xbufs[s % NBUF][...],
            dimension_numbers=(((0,), (1,)), ((), ())),
            preferred_element_type=jnp.float32)
        out_copy(s).start()
        if s + NBUF < STEPS:
            in_copy(s + NBUF).start()

    for s in range(STEPS - OB, STEPS):
        out_copy(s).wait()


def _impl(tensor, W):
    scratch = (
        [pltpu.VMEM((BT, FEATURES), jnp.float32)] * NBUF
        + [pltpu.VMEM((EMBED, BT), jnp.float32)] * OB
        + [pltpu.SemaphoreType.DMA] * (NBUF + OB)
    )
    out_t = pl.pallas_call(
        _embed_kernel,
        in_specs=[
            pl.BlockSpec(memory_space=pl.ANY),
            pl.BlockSpec(memory_space=pltpu.VMEM),
        ],
        out_specs=pl.BlockSpec(memory_space=pl.ANY),
        out_shape=jax.ShapeDtypeStruct((GENOMES, EMBED, BATCH), jnp.float32),
        scratch_shapes=scratch,
    )(tensor, W)
    return out_t.transpose(0, 2, 1)


kernel = jax.jit(_impl)
